# baseline (speedup 1.0000x reference)
import functools

import jax
import jax.numpy as jnp
from jax import lax
from jax.experimental import pallas as pl
from jax.experimental.pallas import tpu as pltpu
from jax.experimental.pallas import tpu_sc as plsc

P = 50000
B = 2
C = 64
GX = 512
GY = 512
NCELL = GX * GY

NC, NS, L = 2, 16, 16
NWORK = NC * NS
PER_W = 1568
GRP = PER_W // L

XB = 64


def _zero_mask():
    def body(o_ref):
        o_ref[...] = jnp.zeros_like(o_ref)

    return pl.pallas_call(
        body,
        out_shape=jax.ShapeDtypeStruct((GX, GY), jnp.int32),
        out_specs=pl.BlockSpec((GX, GY), lambda: (0, 0)),
    )()


_mesh = plsc.VectorSubcoreMesh(core_axis_name="c", subcore_axis_name="s")


def _make_sc_scatter(h):
    @functools.partial(
        pl.kernel,
        out_type=jax.ShapeDtypeStruct((2 * NCELL, C), jnp.float32),
        mesh=_mesh,
        compiler_params=pltpu.CompilerParams(use_tc_tiling_on_sc=False),
        scratch_types=[
            pltpu.VMEM((PER_W,), jnp.int32),
            pltpu.VMEM((PER_W,), jnp.int32),
            pltpu.VMEM((PER_W,), jnp.int32),
            pltpu.VMEM((PER_W,), jnp.int32),
            pltpu.VMEM((PER_W,), jnp.int32),
            pltpu.VMEM((PER_W,), jnp.int32),
            pltpu.VMEM((PER_W, C), jnp.float32),
            pltpu.SemaphoreType.DMA,
            pltpu.SemaphoreType.DMA,
        ],
        name=f"sc_scatter_b{h}",
    )
    def _sc_scatter(feat_hbm, ct_hbm, mask_ref, t2_ref,
                    b_v, x_v, y_v, q_v, q2_v, ones_v, feat_v, sem_t, sem_m):
        wid = lax.axis_index("s") * NC + lax.axis_index("c")
        p0 = jnp.minimum(wid * PER_W, P - PER_W)
        cp_feat = pltpu.async_copy(feat_hbm.at[pl.ds(p0, PER_W)], feat_v, sem_t)
        pltpu.sync_copy(ct_hbm.at[0, pl.ds(p0, PER_W)], b_v)
        pltpu.sync_copy(ct_hbm.at[1, pl.ds(p0, PER_W)], x_v)
        pltpu.sync_copy(ct_hbm.at[2, pl.ds(p0, PER_W)], y_v)

        def build(g, carry):
            sl = pl.ds(g * L, L)
            mine = b_v[sl] == h
            q = x_v[sl] * GY + y_v[sl]
            neg1 = jnp.full((L,), -1, jnp.int32)
            q_v[sl] = jnp.where(mine, q, neg1)
            q2_v[sl] = jnp.where(mine, q + q, neg1)
            ones_v[sl] = jnp.ones((L,), jnp.int32)
            return carry

        lax.fori_loop(0, GRP, build, 0)
        cp_feat.wait()
        cp_mask = pltpu.async_copy(
            ones_v, mask_ref.at[plsc.Indices(q_v, ignored_value=-1)], sem_m)
        pltpu.async_copy(
            feat_v, t2_ref.at[plsc.Indices(q2_v, ignored_value=-1)], sem_t
        ).wait()
        cp_mask.wait()

    return _sc_scatter


_sc_scatter_b = [_make_sc_scatter(0), _make_sc_scatter(1)]


def _transpose_masked(h, mask2d, t128, prev=None):
    def body(*refs):
        if prev is None:
            m_ref, t_ref, o_ref = refs
        else:
            m_ref, t_ref, _, o_ref = refs
        tt = jnp.transpose(t_ref[:, :C], (1, 0))
        m = m_ref[...].reshape(1, XB, GY)
        o_ref[...] = jnp.where(m != 0, tt.reshape(C, XB, GY), 0.0)[None]

    in_specs = [
        pl.BlockSpec((XB, GY), lambda g: (g, 0)),
        pl.BlockSpec((XB * GY, 2 * C), lambda g: (g, 0)),
    ]
    args = [mask2d, t128]
    aliases = {}
    if prev is not None:
        in_specs.append(pl.BlockSpec(memory_space=pl.ANY))
        args.append(prev)
        aliases = {2: 0}
    return pl.pallas_call(
        body,
        grid=(GX // XB,),
        in_specs=in_specs,
        out_specs=pl.BlockSpec(
            (1, C, XB, GY), lambda g: (h, 0, g, 0),
        ),
        out_shape=jax.ShapeDtypeStruct((B, C, GX, GY), jnp.float32),
        input_output_aliases=aliases,
    )(*args)


def kernel(pillar_features, pillar_coords, batch_size):
    del batch_size
    coords_t = pillar_coords.T
    out = None
    for h in range(B):
        mask_ref = jax.new_ref(_zero_mask().reshape(NCELL))
        t2 = _sc_scatter_b[h](pillar_features, coords_t, mask_ref)
        mask2d = jax.freeze(mask_ref).reshape(GX, GY)
        out = _transpose_masked(h, mask2d, t2.reshape(NCELL, 2 * C), out)
    return out


# speedup vs baseline: 1.0460x; 1.0460x over previous
import functools

import jax
import jax.numpy as jnp
from jax import lax
from jax.experimental import pallas as pl
from jax.experimental.pallas import tpu as pltpu
from jax.experimental.pallas import tpu_sc as plsc

P = 50000
B = 2
C = 64
GX = 512
GY = 512
NCELL = B * GX * GY

NC, NS, L = 2, 16, 16
NWORK = NC * NS
PER_W = 1568
GRP = PER_W // L

XB = 64


def _zero_mask():
    def body(o_ref):
        o_ref[...] = jnp.zeros_like(o_ref)

    return pl.pallas_call(
        body,
        out_shape=jax.ShapeDtypeStruct((B * GX, GY), jnp.int32),
        grid=(2,),
        out_specs=pl.BlockSpec((B * GX // 2, GY), lambda i: (i, 0)),
    )()


_mesh = plsc.VectorSubcoreMesh(core_axis_name="c", subcore_axis_name="s")


@functools.partial(
    pl.kernel,
    out_type=jax.ShapeDtypeStruct((2 * NCELL, C), jnp.float32),
    mesh=_mesh,
    compiler_params=pltpu.CompilerParams(use_tc_tiling_on_sc=False),
    scratch_types=[
        pltpu.VMEM((PER_W,), jnp.int32),
        pltpu.VMEM((PER_W,), jnp.int32),
        pltpu.VMEM((PER_W,), jnp.int32),
        pltpu.VMEM((PER_W // 2,), jnp.int32),
        pltpu.VMEM((PER_W // 2,), jnp.int32),
        pltpu.VMEM((PER_W // 2,), jnp.int32),
        pltpu.VMEM((PER_W // 2,), jnp.int32),
        pltpu.VMEM((PER_W // 2,), jnp.int32),
        pltpu.VMEM((PER_W, C), jnp.float32),
        pltpu.SemaphoreType.DMA,
        pltpu.SemaphoreType.DMA,
        pltpu.SemaphoreType.DMA,
        pltpu.SemaphoreType.DMA,
    ],
)
def _sc_scatter(feat_hbm, ct_hbm, mask_ref, t2_ref,
                b_v, x_v, y_v, qa_v, qb_v, qa2_v, qb2_v, ones_v, feat_v,
                sem_ta, sem_tb, sem_ma, sem_mb):
    wid = lax.axis_index("s") * NC + lax.axis_index("c")
    p0 = jnp.minimum(wid * PER_W, P - PER_W)
    half = PER_W // 2
    cp_feat = pltpu.async_copy(feat_hbm.at[pl.ds(p0, PER_W)], feat_v, sem_ta)
    pltpu.sync_copy(ct_hbm.at[0, pl.ds(p0, PER_W)], b_v)
    pltpu.sync_copy(ct_hbm.at[1, pl.ds(p0, PER_W)], x_v)
    pltpu.sync_copy(ct_hbm.at[2, pl.ds(p0, PER_W)], y_v)

    def mkbuild(q_dst, q2_dst, off):
        def build(g, carry):
            sl = pl.ds(off + g * L, L)
            q = b_v[sl] * (GX * GY) + x_v[sl] * GY + y_v[sl]
            dl = pl.ds(g * L, L)
            q_dst[dl] = q
            q2_dst[dl] = q + q
            return carry
        return build

    lax.fori_loop(0, GRP // 2, mkbuild(qa_v, qa2_v, 0), 0)
    lax.fori_loop(0, GRP // 2, mkbuild(qb_v, qb2_v, half), 0)

    def fill_ones(g, carry):
        ones_v[pl.ds(g * L, L)] = jnp.ones((L,), jnp.int32)
        return carry

    lax.fori_loop(0, GRP // 2, fill_ones, 0)
    cp_feat.wait()
    cp_ma = pltpu.async_copy(ones_v, mask_ref.at[qa_v], sem_ma)
    cp_mb = pltpu.async_copy(ones_v, mask_ref.at[qb_v], sem_mb)
    cp_tb = pltpu.async_copy(
        feat_v.at[pl.ds(half, half)], t2_ref.at[qb2_v], sem_tb)
    pltpu.async_copy(
        feat_v.at[pl.ds(0, half)], t2_ref.at[qa2_v], sem_ta).wait()
    cp_tb.wait()
    cp_ma.wait()
    cp_mb.wait()


def _transpose_masked(mask2d, t128):
    def body(m_ref, t_ref, o_ref):
        tt = jnp.transpose(t_ref[:, :C], (1, 0))
        m = m_ref[...].reshape(1, XB, GY)
        o_ref[...] = jnp.where(m != 0, tt.reshape(C, XB, GY), 0.0)[None]

    return pl.pallas_call(
        body,
        grid=(B * GX // XB,),
        in_specs=[
            pl.BlockSpec((XB, GY), lambda g: (g, 0)),
            pl.BlockSpec((XB * GY, 2 * C), lambda g: (g, 0)),
        ],
        out_specs=pl.BlockSpec(
            (1, C, XB, GY),
            lambda g: (g // (GX // XB), 0, g % (GX // XB), 0),
        ),
        out_shape=jax.ShapeDtypeStruct((B, C, GX, GY), jnp.float32),
    )(mask2d, t128)


def kernel(pillar_features, pillar_coords, batch_size):
    del batch_size
    coords_t = pillar_coords.T
    mask_ref = jax.new_ref(_zero_mask().reshape(NCELL))
    t2 = _sc_scatter(pillar_features, coords_t, mask_ref)
    mask2d = jax.freeze(mask_ref).reshape(B * GX, GY)
    return _transpose_masked(mask2d, t2.reshape(NCELL, 2 * C))


# speedup vs baseline: 1.0470x; 1.0009x over previous
import functools

import jax
import jax.numpy as jnp
from jax import lax
from jax.experimental import pallas as pl
from jax.experimental.pallas import tpu as pltpu
from jax.experimental.pallas import tpu_sc as plsc

P = 50000
B = 2
C = 64
GX = 512
GY = 512
NCELL = B * GX * GY

NC, NS, L = 2, 16, 16
NWORK = NC * NS
PER_W = 1568
GRP = PER_W // L

XB = 64


def _zero_mask():
    def body(o_ref):
        o_ref[...] = jnp.zeros_like(o_ref)

    return pl.pallas_call(
        body,
        out_shape=jax.ShapeDtypeStruct((B * GX, GY), jnp.int32),
        grid=(2,),
        out_specs=pl.BlockSpec((B * GX // 2, GY), lambda i: (i, 0)),
    )()


_mesh = plsc.VectorSubcoreMesh(core_axis_name="c", subcore_axis_name="s")


@functools.partial(
    pl.kernel,
    out_type=jax.ShapeDtypeStruct((2 * NCELL, C), jnp.float32),
    mesh=_mesh,
    compiler_params=pltpu.CompilerParams(use_tc_tiling_on_sc=False),
    scratch_types=[
        pltpu.VMEM((PER_W,), jnp.int32),
        pltpu.VMEM((PER_W,), jnp.int32),
        pltpu.VMEM((PER_W,), jnp.int32),
        pltpu.VMEM((PER_W,), jnp.int32),
        pltpu.VMEM((PER_W,), jnp.int32),
        pltpu.VMEM((PER_W,), jnp.int32),
        pltpu.VMEM((PER_W, C), jnp.float32),
        pltpu.SemaphoreType.DMA,
        pltpu.SemaphoreType.DMA,
    ],
)
def _sc_scatter(feat_hbm, ct_hbm, mask_ref, t2_ref,
                b_v, x_v, y_v, q_v, q2_v, ones_v, feat_v, sem_t, sem_m):
    wid = lax.axis_index("s") * NC + lax.axis_index("c")
    p0 = jnp.minimum(wid * PER_W, P - PER_W)
    cp_feat = pltpu.async_copy(feat_hbm.at[pl.ds(p0, PER_W)], feat_v, sem_t)
    pltpu.sync_copy(ct_hbm.at[0, pl.ds(p0, PER_W)], b_v)
    pltpu.sync_copy(ct_hbm.at[1, pl.ds(p0, PER_W)], x_v)
    pltpu.sync_copy(ct_hbm.at[2, pl.ds(p0, PER_W)], y_v)

    def build(g, carry):
        sl = pl.ds(g * L, L)
        q = b_v[sl] * (GX * GY) + x_v[sl] * GY + y_v[sl]
        q_v[sl] = q
        q2_v[sl] = q + q
        ones_v[sl] = jnp.ones((L,), jnp.int32)
        return carry

    lax.fori_loop(0, GRP, build, 0)
    cp_feat.wait()
    cp_mask = pltpu.async_copy(ones_v, mask_ref.at[q_v], sem_m)
    pltpu.async_copy(feat_v, t2_ref.at[q2_v], sem_t).wait()
    cp_mask.wait()


def _transpose_masked(mask2d, t128):
    def body(m_ref, t_ref, o_ref):
        tt = jnp.transpose(t_ref[:, :C], (1, 0))
        m = m_ref[...].reshape(1, XB, GY)
        o_ref[...] = jnp.where(m != 0, tt.reshape(C, XB, GY), 0.0)[None]

    return pl.pallas_call(
        body,
        grid=(B * GX // XB,),
        in_specs=[
            pl.BlockSpec((XB, GY), lambda g: (g, 0)),
            pl.BlockSpec((XB * GY, 2 * C), lambda g: (g, 0)),
        ],
        out_specs=pl.BlockSpec(
            (1, C, XB, GY),
            lambda g: (g // (GX // XB), 0, g % (GX // XB), 0),
        ),
        out_shape=jax.ShapeDtypeStruct((B, C, GX, GY), jnp.float32),
    )(mask2d, t128)


def kernel(pillar_features, pillar_coords, batch_size):
    del batch_size
    coords_t = pillar_coords.T
    mask_ref = jax.new_ref(_zero_mask().reshape(NCELL))
    t2 = _sc_scatter(pillar_features, coords_t, mask_ref)
    mask2d = jax.freeze(mask_ref).reshape(B * GX, GY)
    return _transpose_masked(mask2d, t2.reshape(NCELL, 2 * C))
